# Initial kernel scaffold; baseline (speedup 1.0000x reference)
import jax
import jax.numpy as jnp
from jax.experimental import pallas as pl


def kernel(x, lin_w0, lin_b0, lin_w1, lin_b1, lin_w2, lin_b2, lin_w3, lin_b3, lin_w4, lin_b4, conv_w0, conv_b0, conv_w1, conv_b1, conv_w2, conv_b2, conv_w3, conv_b3, conv_w4, conv_b4, conv_w5, conv_b5):
    raise NotImplementedError("write your pallas kernel here")


# speedup vs baseline: 1.3174x; 1.3174x over previous
import functools

import jax
import jax.numpy as jnp
from jax.experimental import pallas as pl
from jax.experimental.pallas import tpu as pltpu

_CONV_KS = (7, 5, 3, 1, 1, 1)
_C = 16

_ROWS, _COLS = 64, 64
_MR, _MC, _GUARD = 4, 8, 256
_WROW = _COLS + 2 * _MC
_HPAD = _ROWS + 2 * _MR
_L = _HPAD * _WROW
_LCAN = _L + 2 * _GUARD
_DSTART = _GUARD + _MR * _WROW
_LOUT = _ROWS * _WROW


def _mlp_kernel(x_ref, w0, b0, w1, b1, w2, b2, w3, b3, w4c, b4s, o_ref, *, nslab):
    for s in range(nslab):
        h = x_ref[0, s]
        for w, b in ((w0, b0), (w1, b1), (w2, b2), (w3, b3)):
            h = jnp.dot(w[...], h, preferred_element_type=jnp.float32) + b[...]
            h = jnp.maximum(h, 0.0)
        o = jnp.sum(h * w4c[...], axis=0, keepdims=True) + b4s[...]
        o_ref[0, s] = jnp.maximum(o, 0.0)[0]


def _conv_chain_kernel(can_ref, *refs, ks):
    o_ref = refs[-1]
    nlayer = len(ks)
    col = jax.lax.broadcasted_iota(jnp.int32, (1, _LOUT), 1) % _WROW
    colmask = jnp.logical_and(col >= _MC, col < _MC + _COLS)

    can = can_ref[0]
    for layer in range(nlayer):
        k = ks[layer]
        p = (k - 1) // 2
        w_all = refs[2 * layer][...]
        b = refs[2 * layer + 1][...]
        if k == 1:
            sl = can[:, _DSTART:_DSTART + _LOUT]
            acc = jnp.dot(w_all, sl, preferred_element_type=jnp.float32)
        else:
            xs = jnp.concatenate(
                [can[:, _GUARD + j - p:_GUARD + j - p + _L] for j in range(k)],
                axis=0)
            y = jnp.dot(w_all, xs, preferred_element_type=jnp.float32)
            base = _MR * _WROW
            acc = y[0 * _C:1 * _C, base + (0 - p) * _WROW:][:, :_LOUT]
            for i in range(1, k):
                off = base + (i - p) * _WROW
                acc = acc + y[i * _C:(i + 1) * _C, off:off + _LOUT]
        acc = jnp.maximum(acc + b, 0.0)
        if layer + 1 < nlayer:
            acc = jnp.where(colmask, acc, 0.0)
            zg = jnp.zeros((_C, _DSTART), jnp.float32)
            can = jnp.concatenate([zg, acc, zg], axis=1)
        else:
            o_ref[0] = acc


def _run_mlp(x, lin_params):
    N, C, D, H, W = x.shape
    P = H * W
    NC = N * C
    S = 2
    xg = x.reshape(NC // S, S, D, P)

    args = []
    for w, b in lin_params[:4]:
        args.append(w)
        args.append(b.reshape(-1, 1))
    w4, b4 = lin_params[4]
    args.append(w4.reshape(-1, 1))
    args.append(b4.reshape(1, 1))

    flops = 2 * NC * P * sum(w.shape[0] * w.shape[1] for w, _ in lin_params)
    bytes_accessed = 4 * (x.size + sum(int(a.size) for a in args) + NC * P)

    out = pl.pallas_call(
        functools.partial(_mlp_kernel, nslab=S),
        out_shape=jax.ShapeDtypeStruct((NC // S, S, P), jnp.float32),
        grid_spec=pltpu.PrefetchScalarGridSpec(
            num_scalar_prefetch=0,
            grid=(NC // S,),
            in_specs=[pl.BlockSpec((1, S, D, P), lambda g: (g, 0, 0, 0))]
            + [pl.BlockSpec(a.shape, lambda g: (0, 0)) for a in args],
            out_specs=pl.BlockSpec((1, S, P), lambda g: (g, 0, 0)),
        ),
        compiler_params=pltpu.CompilerParams(dimension_semantics=("parallel",)),
        cost_estimate=pl.CostEstimate(
            flops=flops, transcendentals=0, bytes_accessed=bytes_accessed),
    )(xg, *args)
    return out.reshape(N, C, H, W)


def _run_convs(h_img, conv_params):
    N = h_img.shape[0]
    can = jnp.pad(h_img, ((0, 0), (0, 0), (_MR, _MR), (_MC, _MC)))
    can = can.reshape(N, _C, _L)
    can = jnp.pad(can, ((0, 0), (0, 0), (_GUARD, _GUARD)))

    args = []
    flops = 0
    for (w, b), k in zip(conv_params, _CONV_KS):
        w_all = jnp.transpose(w, (2, 0, 3, 1)).reshape(k * _C, k * _C)
        args.append(w_all)
        args.append(b.reshape(_C, 1))
        flops += 2 * N * _LOUT * _C * _C * k * k
    bytes_accessed = 4 * (can.size + sum(int(a.size) for a in args)
                          + N * _C * _LOUT)

    out = pl.pallas_call(
        functools.partial(_conv_chain_kernel, ks=_CONV_KS),
        out_shape=jax.ShapeDtypeStruct((N, _C, _LOUT), jnp.float32),
        grid_spec=pltpu.PrefetchScalarGridSpec(
            num_scalar_prefetch=0,
            grid=(N,),
            in_specs=[pl.BlockSpec((1, _C, _LCAN), lambda n: (n, 0, 0))]
            + [pl.BlockSpec(a.shape, lambda n: (0, 0)) for a in args],
            out_specs=pl.BlockSpec((1, _C, _LOUT), lambda n: (n, 0, 0)),
        ),
        compiler_params=pltpu.CompilerParams(dimension_semantics=("parallel",)),
        cost_estimate=pl.CostEstimate(
            flops=flops, transcendentals=0, bytes_accessed=bytes_accessed),
    )(can, *args)

    out = out.reshape(N, _C, _ROWS, _WROW)[..., _MC:_MC + _COLS]
    return out


@jax.jit
def _forward(x, lin_params, conv_params):
    h = _run_mlp(x, lin_params)
    return _run_convs(h, conv_params)


def kernel(x, lin_w0, lin_b0, lin_w1, lin_b1, lin_w2, lin_b2, lin_w3, lin_b3,
           lin_w4, lin_b4, conv_w0, conv_b0, conv_w1, conv_b1, conv_w2,
           conv_b2, conv_w3, conv_b3, conv_w4, conv_b4, conv_w5, conv_b5):
    lin_params = [(lin_w0, lin_b0), (lin_w1, lin_b1), (lin_w2, lin_b2),
                  (lin_w3, lin_b3), (lin_w4, lin_b4)]
    conv_params = [(conv_w0, conv_b0), (conv_w1, conv_b1), (conv_w2, conv_b2),
                   (conv_w3, conv_b3), (conv_w4, conv_b4), (conv_w5, conv_b5)]
    return _forward(x, lin_params, conv_params)


# speedup vs baseline: 1.4240x; 1.0809x over previous
import functools

import jax
import jax.numpy as jnp
from jax.experimental import pallas as pl
from jax.experimental.pallas import tpu as pltpu

_CONV_KS = (7, 5, 3, 1, 1, 1)
_C = 16

_ROWS, _COLS = 64, 64
_MR, _MC, _GUARD = 4, 8, 256
_WROW = _COLS + 2 * _MC
_HPAD = _ROWS + 2 * _MR
_L = _HPAD * _WROW
_LCAN = _L + 2 * _GUARD
_DSTART = _GUARD + _MR * _WROW
_LOUT = _ROWS * _WROW


def _mlp_kernel(x_ref, w0, b0, w1, b1, w2, b2, w3, b3, w4c, b4s, o_ref, *, nslab):
    for s in range(nslab):
        h = x_ref[s]
        for w, b in ((w0, b0), (w1, b1), (w2, b2), (w3, b3)):
            h = jnp.dot(w[...], h, preferred_element_type=jnp.float32) + b[...]
            h = jnp.maximum(h, 0.0)
        o = jnp.sum(h * w4c[...], axis=0, keepdims=True) + b4s[...]
        o_ref[s] = jnp.maximum(o, 0.0)[0]


def _conv_chain_kernel(img_ref, *refs, ks):
    o_ref = refs[-1]
    nlayer = len(ks)
    col = jax.lax.broadcasted_iota(jnp.int32, (1, _LOUT), 1) % _WROW
    colmask = jnp.logical_and(col >= _MC, col < _MC + _COLS)

    img = img_ref[...]
    z8 = jnp.zeros((_C, _MC), jnp.float32)
    pieces = []
    for r in range(_ROWS):
        pieces += [z8, img[:, r * _COLS:(r + 1) * _COLS], z8]
    flat = jnp.concatenate(pieces, axis=1)
    zg = jnp.zeros((_C, _DSTART), jnp.float32)
    can = jnp.concatenate([zg, flat, zg], axis=1)

    for layer in range(nlayer):
        k = ks[layer]
        p = (k - 1) // 2
        w_all = refs[2 * layer][...]
        b = refs[2 * layer + 1][...]
        if k == 1:
            sl = can[:, _DSTART:_DSTART + _LOUT]
            acc = jnp.dot(w_all, sl, preferred_element_type=jnp.float32)
        else:
            xs = jnp.concatenate(
                [can[:, _GUARD + j - p:_GUARD + j - p + _L] for j in range(k)],
                axis=0)
            y = jnp.dot(w_all, xs, preferred_element_type=jnp.float32)
            base = _MR * _WROW
            acc = y[0 * _C:1 * _C, base + (0 - p) * _WROW:][:, :_LOUT]
            for i in range(1, k):
                off = base + (i - p) * _WROW
                acc = acc + y[i * _C:(i + 1) * _C, off:off + _LOUT]
        acc = jnp.maximum(acc + b, 0.0)
        if layer + 1 < nlayer:
            acc = jnp.where(colmask, acc, 0.0)
            can = jnp.concatenate([zg, acc, zg], axis=1)
        else:
            o_ref[0] = acc.reshape(_C, _ROWS, _WROW)[:, :, _MC:_MC + _COLS]


def _run_mlp(x, lin_params):
    N, C, D, H, W = x.shape
    P = H * W
    NC = N * C
    S = 8
    x_flat = x.reshape(NC, D, P)

    args = []
    for w, b in lin_params[:4]:
        args.append(w)
        args.append(b.reshape(-1, 1))
    w4, b4 = lin_params[4]
    args.append(w4.reshape(-1, 1))
    args.append(b4.reshape(1, 1))

    flops = 2 * NC * P * sum(w.shape[0] * w.shape[1] for w, _ in lin_params)
    bytes_accessed = 4 * (x.size + sum(int(a.size) for a in args) + NC * P)

    out = pl.pallas_call(
        functools.partial(_mlp_kernel, nslab=S),
        out_shape=jax.ShapeDtypeStruct((NC, P), jnp.float32),
        grid_spec=pltpu.PrefetchScalarGridSpec(
            num_scalar_prefetch=0,
            grid=(NC // S,),
            in_specs=[pl.BlockSpec((S, D, P), lambda g: (g, 0, 0))]
            + [pl.BlockSpec(a.shape, lambda g: (0, 0)) for a in args],
            out_specs=pl.BlockSpec((S, P), lambda g: (g, 0)),
        ),
        compiler_params=pltpu.CompilerParams(dimension_semantics=("parallel",)),
        cost_estimate=pl.CostEstimate(
            flops=flops, transcendentals=0, bytes_accessed=bytes_accessed),
    )(x_flat, *args)
    return out


def _run_convs(h_flat, conv_params, N):
    args = []
    flops = 0
    for (w, b), k in zip(conv_params, _CONV_KS):
        w_all = jnp.transpose(w, (2, 0, 3, 1)).reshape(k * _C, k * _C)
        args.append(w_all)
        args.append(b.reshape(_C, 1))
        flops += 2 * N * _LOUT * _C * _C * k * k
    bytes_accessed = 4 * (h_flat.size + sum(int(a.size) for a in args)
                          + N * _C * _ROWS * _COLS)

    out = pl.pallas_call(
        functools.partial(_conv_chain_kernel, ks=_CONV_KS),
        out_shape=jax.ShapeDtypeStruct((N, _C, _ROWS, _COLS), jnp.float32),
        grid_spec=pltpu.PrefetchScalarGridSpec(
            num_scalar_prefetch=0,
            grid=(N,),
            in_specs=[pl.BlockSpec((_C, 4096), lambda n: (n, 0))]
            + [pl.BlockSpec(a.shape, lambda n: (0, 0)) for a in args],
            out_specs=pl.BlockSpec((1, _C, _ROWS, _COLS), lambda n: (n, 0, 0, 0)),
        ),
        compiler_params=pltpu.CompilerParams(dimension_semantics=("parallel",)),
        cost_estimate=pl.CostEstimate(
            flops=flops, transcendentals=0, bytes_accessed=bytes_accessed),
    )(h_flat, *args)
    return out


@jax.jit
def _forward(x, lin_params, conv_params):
    h = _run_mlp(x, lin_params)
    return _run_convs(h, conv_params, x.shape[0])


def kernel(x, lin_w0, lin_b0, lin_w1, lin_b1, lin_w2, lin_b2, lin_w3, lin_b3,
           lin_w4, lin_b4, conv_w0, conv_b0, conv_w1, conv_b1, conv_w2,
           conv_b2, conv_w3, conv_b3, conv_w4, conv_b4, conv_w5, conv_b5):
    lin_params = [(lin_w0, lin_b0), (lin_w1, lin_b1), (lin_w2, lin_b2),
                  (lin_w3, lin_b3), (lin_w4, lin_b4)]
    conv_params = [(conv_w0, conv_b0), (conv_w1, conv_b1), (conv_w2, conv_b2),
                   (conv_w3, conv_b3), (conv_w4, conv_b4), (conv_w5, conv_b5)]
    return _forward(x, lin_params, conv_params)


# speedup vs baseline: 1.5984x; 1.1225x over previous
import functools

import jax
import jax.numpy as jnp
from jax.experimental import pallas as pl
from jax.experimental.pallas import tpu as pltpu

_CONV_KS = (7, 5, 3, 1, 1, 1)
_C = 16

_ROWS, _COLS = 64, 64
_MR, _MC, _GUARD = 4, 8, 256
_WROW = _COLS + 2 * _MC
_HPAD = _ROWS + 2 * _MR
_L = _HPAD * _WROW
_LCAN = _L + 2 * _GUARD
_DSTART = _GUARD + _MR * _WROW
_LOUT = _ROWS * _WROW


def _mlp_kernel(x_ref, w0, b0, w1, b1, w2, b2, w3, b3, w4c, b4s, o_ref, *, nslab):
    for s in range(nslab):
        h = x_ref[s].reshape(x_ref.shape[1], -1)
        for w, b in ((w0, b0), (w1, b1), (w2, b2), (w3, b3)):
            h = jnp.dot(w[...], h, preferred_element_type=jnp.float32) + b[...]
            h = jnp.maximum(h, 0.0)
        o = jnp.sum(h * w4c[...], axis=0, keepdims=True) + b4s[...]
        o_ref[s] = jnp.maximum(o, 0.0)[0]


def _conv_chain_kernel(img_ref, *refs, ks):
    o_ref = refs[-1]
    nlayer = len(ks)
    col = jax.lax.broadcasted_iota(jnp.int32, (1, _LOUT), 1) % _WROW
    colmask = jnp.logical_and(col >= _MC, col < _MC + _COLS)

    img = img_ref[...]
    z8 = jnp.zeros((_C, _MC), jnp.float32)
    pieces = []
    for r in range(_ROWS):
        pieces += [z8, img[:, r * _COLS:(r + 1) * _COLS], z8]
    flat = jnp.concatenate(pieces, axis=1)
    zg = jnp.zeros((_C, _DSTART), jnp.float32)
    can = jnp.concatenate([zg, flat, zg], axis=1)

    for layer in range(nlayer):
        k = ks[layer]
        p = (k - 1) // 2
        w_all = refs[2 * layer][...]
        b = refs[2 * layer + 1][...]
        if k == 1:
            sl = can[:, _DSTART:_DSTART + _LOUT]
            acc = jnp.dot(w_all, sl, preferred_element_type=jnp.float32)
        else:
            xs = jnp.concatenate(
                [can[:, _GUARD + j - p:_GUARD + j - p + _L] for j in range(k)],
                axis=0)
            y = jnp.dot(w_all, xs, preferred_element_type=jnp.float32)
            base = _MR * _WROW
            acc = y[0 * _C:1 * _C, base + (0 - p) * _WROW:][:, :_LOUT]
            for i in range(1, k):
                off = base + (i - p) * _WROW
                acc = acc + y[i * _C:(i + 1) * _C, off:off + _LOUT]
        acc = jnp.maximum(acc + b, 0.0)
        if layer + 1 < nlayer:
            acc = jnp.where(colmask, acc, 0.0)
            can = jnp.concatenate([zg, acc, zg], axis=1)
        else:
            o_ref[0] = acc.reshape(_C, _ROWS, _WROW)[:, :, _MC:_MC + _COLS]


def _run_mlp(x, lin_params):
    N, C, D, H, W = x.shape
    P = H * W
    NC = N * C
    S = 8
    x_flat = x.reshape(NC, D, H, W)

    args = []
    for w, b in lin_params[:4]:
        args.append(w)
        args.append(b.reshape(-1, 1))
    w4, b4 = lin_params[4]
    args.append(w4.reshape(-1, 1))
    args.append(b4.reshape(1, 1))

    flops = 2 * NC * P * sum(w.shape[0] * w.shape[1] for w, _ in lin_params)
    bytes_accessed = 4 * (x.size + sum(int(a.size) for a in args) + NC * P)

    out = pl.pallas_call(
        functools.partial(_mlp_kernel, nslab=S),
        out_shape=jax.ShapeDtypeStruct((NC, P), jnp.float32),
        grid_spec=pltpu.PrefetchScalarGridSpec(
            num_scalar_prefetch=0,
            grid=(NC // S,),
            in_specs=[pl.BlockSpec((S, D, H, W), lambda g: (g, 0, 0, 0))]
            + [pl.BlockSpec(a.shape, lambda g: (0, 0)) for a in args],
            out_specs=pl.BlockSpec((S, P), lambda g: (g, 0)),
        ),
        compiler_params=pltpu.CompilerParams(dimension_semantics=("parallel",)),
        cost_estimate=pl.CostEstimate(
            flops=flops, transcendentals=0, bytes_accessed=bytes_accessed),
    )(x_flat, *args)
    return out


def _run_convs(h_flat, conv_params, N):
    args = []
    flops = 0
    for (w, b), k in zip(conv_params, _CONV_KS):
        w_all = jnp.transpose(w, (2, 0, 3, 1)).reshape(k * _C, k * _C)
        args.append(w_all)
        args.append(b.reshape(_C, 1))
        flops += 2 * N * _LOUT * _C * _C * k * k
    bytes_accessed = 4 * (h_flat.size + sum(int(a.size) for a in args)
                          + N * _C * _ROWS * _COLS)

    out = pl.pallas_call(
        functools.partial(_conv_chain_kernel, ks=_CONV_KS),
        out_shape=jax.ShapeDtypeStruct((N, _C, _ROWS, _COLS), jnp.float32),
        grid_spec=pltpu.PrefetchScalarGridSpec(
            num_scalar_prefetch=0,
            grid=(N,),
            in_specs=[pl.BlockSpec((_C, 4096), lambda n: (n, 0))]
            + [pl.BlockSpec(a.shape, lambda n: (0, 0)) for a in args],
            out_specs=pl.BlockSpec((1, _C, _ROWS, _COLS), lambda n: (n, 0, 0, 0)),
        ),
        compiler_params=pltpu.CompilerParams(dimension_semantics=("parallel",)),
        cost_estimate=pl.CostEstimate(
            flops=flops, transcendentals=0, bytes_accessed=bytes_accessed),
    )(h_flat, *args)
    return out


@jax.jit
def _forward(x, lin_params, conv_params):
    h = _run_mlp(x, lin_params)
    return _run_convs(h, conv_params, x.shape[0])


def kernel(x, lin_w0, lin_b0, lin_w1, lin_b1, lin_w2, lin_b2, lin_w3, lin_b3,
           lin_w4, lin_b4, conv_w0, conv_b0, conv_w1, conv_b1, conv_w2,
           conv_b2, conv_w3, conv_b3, conv_w4, conv_b4, conv_w5, conv_b5):
    lin_params = [(lin_w0, lin_b0), (lin_w1, lin_b1), (lin_w2, lin_b2),
                  (lin_w3, lin_b3), (lin_w4, lin_b4)]
    conv_params = [(conv_w0, conv_b0), (conv_w1, conv_b1), (conv_w2, conv_b2),
                   (conv_w3, conv_b3), (conv_w4, conv_b4), (conv_w5, conv_b5)]
    return _forward(x, lin_params, conv_params)


# speedup vs baseline: 1.6158x; 1.0109x over previous
import functools

import jax
import jax.numpy as jnp
from jax.experimental import pallas as pl
from jax.experimental.pallas import tpu as pltpu

_CONV_KS = (7, 5, 3, 1, 1, 1)
_C = 16

_ROWS, _COLS = 64, 64
_MR, _MC, _GUARD = 4, 8, 256
_WROW = _COLS + 2 * _MC
_HPAD = _ROWS + 2 * _MR
_L = _HPAD * _WROW
_LCAN = _L + 2 * _GUARD
_DSTART = _GUARD + _MR * _WROW
_LOUT = _ROWS * _WROW


def _mlp_kernel(x_ref, w0, b0, w1, b1, w2, b2, w3, b3, w4c, b4s, o_ref, *, nslab):
    D = x_ref.shape[1]
    P = x_ref.shape[2] * x_ref.shape[3]
    for t in range(nslab // 2):
        h = jnp.concatenate(
            [x_ref[2 * t].reshape(D, P), x_ref[2 * t + 1].reshape(D, P)],
            axis=1)
        for w, b in ((w0, b0), (w1, b1), (w2, b2), (w3, b3)):
            h = jnp.dot(w[...], h, preferred_element_type=jnp.float32) + b[...]
            h = jnp.maximum(h, 0.0)
        o = jnp.sum(h * w4c[...], axis=0, keepdims=True) + b4s[...]
        o = jnp.maximum(o, 0.0)
        o_ref[2 * t] = o[0, :P]
        o_ref[2 * t + 1] = o[0, P:]


def _conv_chain_kernel(img_ref, *refs, ks):
    o_ref = refs[-1]
    nlayer = len(ks)
    col = jax.lax.broadcasted_iota(jnp.int32, (1, _LOUT), 1) % _WROW
    colmask = jnp.logical_and(col >= _MC, col < _MC + _COLS)

    img = img_ref[...].astype(jnp.bfloat16)
    z8 = jnp.zeros((_C, _MC), jnp.bfloat16)
    pieces = []
    for r in range(_ROWS):
        pieces += [z8, img[:, r * _COLS:(r + 1) * _COLS], z8]
    flat = jnp.concatenate(pieces, axis=1)
    zg = jnp.zeros((_C, _DSTART), jnp.bfloat16)
    can = jnp.concatenate([zg, flat, zg], axis=1)

    for layer in range(nlayer):
        k = ks[layer]
        p = (k - 1) // 2
        w_all = refs[2 * layer][...]
        b = refs[2 * layer + 1][...]
        if k == 1:
            sl = can[:, _DSTART:_DSTART + _LOUT]
            acc = jnp.dot(w_all, sl, preferred_element_type=jnp.float32)
        else:
            xs = jnp.concatenate(
                [can[:, _GUARD + j - p:_GUARD + j - p + _L] for j in range(k)],
                axis=0)
            y = jnp.dot(w_all, xs, preferred_element_type=jnp.float32)
            base = _MR * _WROW
            acc = y[0 * _C:1 * _C, base + (0 - p) * _WROW:][:, :_LOUT]
            for i in range(1, k):
                off = base + (i - p) * _WROW
                acc = acc + y[i * _C:(i + 1) * _C, off:off + _LOUT]
        acc = jnp.maximum(acc + b, 0.0)
        if layer + 1 < nlayer:
            acc = jnp.where(colmask, acc, 0.0).astype(jnp.bfloat16)
            can = jnp.concatenate([zg, acc, zg], axis=1)
        else:
            o_ref[0] = acc.reshape(_C, _ROWS, _WROW)[:, :, _MC:_MC + _COLS]


def _run_mlp(x, lin_params):
    N, C, D, H, W = x.shape
    P = H * W
    NC = N * C
    S = 8
    x_flat = x.reshape(NC, D, H, W)

    args = []
    for w, b in lin_params[:4]:
        args.append(w)
        args.append(b.reshape(-1, 1))
    w4, b4 = lin_params[4]
    args.append(w4.reshape(-1, 1))
    args.append(b4.reshape(1, 1))

    flops = 2 * NC * P * sum(w.shape[0] * w.shape[1] for w, _ in lin_params)
    bytes_accessed = 4 * (x.size + sum(int(a.size) for a in args) + NC * P)

    out = pl.pallas_call(
        functools.partial(_mlp_kernel, nslab=S),
        out_shape=jax.ShapeDtypeStruct((NC, P), jnp.float32),
        grid_spec=pltpu.PrefetchScalarGridSpec(
            num_scalar_prefetch=0,
            grid=(NC // S,),
            in_specs=[pl.BlockSpec((S, D, H, W), lambda g: (g, 0, 0, 0))]
            + [pl.BlockSpec(a.shape, lambda g: (0, 0)) for a in args],
            out_specs=pl.BlockSpec((S, P), lambda g: (g, 0)),
        ),
        compiler_params=pltpu.CompilerParams(dimension_semantics=("parallel",)),
        cost_estimate=pl.CostEstimate(
            flops=flops, transcendentals=0, bytes_accessed=bytes_accessed),
    )(x_flat, *args)
    return out


def _run_convs(h_flat, conv_params, N):
    args = []
    flops = 0
    for (w, b), k in zip(conv_params, _CONV_KS):
        w_all = jnp.transpose(w, (2, 0, 3, 1)).reshape(k * _C, k * _C)
        w_all = w_all.astype(jnp.bfloat16)
        args.append(w_all)
        args.append(b.reshape(_C, 1))
        flops += 2 * N * _LOUT * _C * _C * k * k
    bytes_accessed = 4 * (h_flat.size + sum(int(a.size) for a in args)
                          + N * _C * _ROWS * _COLS)

    out = pl.pallas_call(
        functools.partial(_conv_chain_kernel, ks=_CONV_KS),
        out_shape=jax.ShapeDtypeStruct((N, _C, _ROWS, _COLS), jnp.float32),
        grid_spec=pltpu.PrefetchScalarGridSpec(
            num_scalar_prefetch=0,
            grid=(N,),
            in_specs=[pl.BlockSpec((_C, 4096), lambda n: (n, 0))]
            + [pl.BlockSpec(a.shape, lambda n: (0, 0)) for a in args],
            out_specs=pl.BlockSpec((1, _C, _ROWS, _COLS), lambda n: (n, 0, 0, 0)),
        ),
        compiler_params=pltpu.CompilerParams(dimension_semantics=("parallel",)),
        cost_estimate=pl.CostEstimate(
            flops=flops, transcendentals=0, bytes_accessed=bytes_accessed),
    )(h_flat, *args)
    return out


@jax.jit
def _forward(x, lin_params, conv_params):
    h = _run_mlp(x, lin_params)
    return _run_convs(h, conv_params, x.shape[0])


def kernel(x, lin_w0, lin_b0, lin_w1, lin_b1, lin_w2, lin_b2, lin_w3, lin_b3,
           lin_w4, lin_b4, conv_w0, conv_b0, conv_w1, conv_b1, conv_w2,
           conv_b2, conv_w3, conv_b3, conv_w4, conv_b4, conv_w5, conv_b5):
    lin_params = [(lin_w0, lin_b0), (lin_w1, lin_b1), (lin_w2, lin_b2),
                  (lin_w3, lin_b3), (lin_w4, lin_b4)]
    conv_params = [(conv_w0, conv_b0), (conv_w1, conv_b1), (conv_w2, conv_b2),
                   (conv_w3, conv_b3), (conv_w4, conv_b4), (conv_w5, conv_b5)]
    return _forward(x, lin_params, conv_params)


# speedup vs baseline: 1.6174x; 1.0010x over previous
import functools

import jax
import jax.numpy as jnp
from jax.experimental import pallas as pl
from jax.experimental.pallas import tpu as pltpu

_CONV_KS = (7, 5, 3, 1, 1, 1)
_C = 16

_ROWS, _COLS = 64, 64
_MR, _MC, _GUARD = 4, 8, 256
_WROW = _COLS + 2 * _MC
_HPAD = _ROWS + 2 * _MR
_L = _HPAD * _WROW
_LCAN = _L + 2 * _GUARD
_DSTART = _GUARD + _MR * _WROW
_LOUT = _ROWS * _WROW


def _mlp_kernel(x_ref, w0, b0, w1, b1, w2, b2, w3, b3, w4c, b4s, o_ref, *, nslab):
    D = x_ref.shape[1]
    P = x_ref.shape[2] * x_ref.shape[3]
    for t in range(nslab // 2):
        h = jnp.concatenate(
            [x_ref[2 * t].reshape(D, P), x_ref[2 * t + 1].reshape(D, P)],
            axis=1)
        for w, b in ((w0, b0), (w1, b1), (w2, b2), (w3, b3)):
            h = jnp.dot(w[...], h, preferred_element_type=jnp.float32) + b[...]
            h = jnp.maximum(h, 0.0)
        o = jnp.sum(h * w4c[...], axis=0, keepdims=True) + b4s[...]
        o = jnp.maximum(o, 0.0)
        o_ref[2 * t] = o[0, :P]
        o_ref[2 * t + 1] = o[0, P:]


def _conv_chain_kernel(img_ref, *refs, ks):
    o_ref = refs[-1]
    nlayer = len(ks)
    col = jax.lax.broadcasted_iota(jnp.int32, (1, _LOUT), 1) % _WROW
    colmask = jnp.logical_and(col >= _MC, col < _MC + _COLS)

    img = img_ref[...]
    z8 = jnp.zeros((_C, _MC), jnp.float32)
    pieces = []
    for r in range(_ROWS):
        pieces += [z8, img[:, r * _COLS:(r + 1) * _COLS], z8]
    flat = jnp.concatenate(pieces, axis=1)
    zg = jnp.zeros((_C, _DSTART), jnp.float32)
    can = jnp.concatenate([zg, flat, zg], axis=1)

    for layer in range(nlayer):
        k = ks[layer]
        p = (k - 1) // 2
        w_all = refs[2 * layer][...]
        b = refs[2 * layer + 1][...]
        if k == 1:
            sl = can[:, _DSTART:_DSTART + _LOUT]
            acc = jnp.dot(w_all, sl, preferred_element_type=jnp.float32)
        else:
            xs = jnp.concatenate(
                [can[:, _GUARD + j - p:_GUARD + j - p + _L] for j in range(k)],
                axis=0)
            y = jnp.dot(w_all, xs, preferred_element_type=jnp.float32)
            base = _MR * _WROW
            acc = y[0 * _C:1 * _C, base + (0 - p) * _WROW:][:, :_LOUT]
            for i in range(1, k):
                off = base + (i - p) * _WROW
                acc = acc + y[i * _C:(i + 1) * _C, off:off + _LOUT]
        acc = jnp.maximum(acc + b, 0.0)
        if layer + 1 < nlayer:
            acc = jnp.where(colmask, acc, 0.0)
            can = jnp.concatenate([zg, acc, zg], axis=1)
        else:
            o_ref[0] = acc.reshape(_C, _ROWS, _WROW)[:, :, _MC:_MC + _COLS]


def _run_mlp(x, lin_params):
    N, C, D, H, W = x.shape
    P = H * W
    NC = N * C
    S = 8
    x_flat = x.reshape(NC, D, H, W)

    args = []
    for w, b in lin_params[:4]:
        args.append(w)
        args.append(b.reshape(-1, 1))
    w4, b4 = lin_params[4]
    args.append(w4.reshape(-1, 1))
    args.append(b4.reshape(1, 1))

    flops = 2 * NC * P * sum(w.shape[0] * w.shape[1] for w, _ in lin_params)
    bytes_accessed = 4 * (x.size + sum(int(a.size) for a in args) + NC * P)

    out = pl.pallas_call(
        functools.partial(_mlp_kernel, nslab=S),
        out_shape=jax.ShapeDtypeStruct((NC, P), jnp.float32),
        grid_spec=pltpu.PrefetchScalarGridSpec(
            num_scalar_prefetch=0,
            grid=(NC // S,),
            in_specs=[pl.BlockSpec((S, D, H, W), lambda g: (g, 0, 0, 0))]
            + [pl.BlockSpec(a.shape, lambda g: (0, 0)) for a in args],
            out_specs=pl.BlockSpec((S, P), lambda g: (g, 0)),
        ),
        compiler_params=pltpu.CompilerParams(dimension_semantics=("parallel",)),
        cost_estimate=pl.CostEstimate(
            flops=flops, transcendentals=0, bytes_accessed=bytes_accessed),
    )(x_flat, *args)
    return out


def _run_convs(h_flat, conv_params, N):
    args = []
    flops = 0
    for (w, b), k in zip(conv_params, _CONV_KS):
        w_all = jnp.transpose(w, (2, 0, 3, 1)).reshape(k * _C, k * _C)
        args.append(w_all)
        args.append(b.reshape(_C, 1))
        flops += 2 * N * _LOUT * _C * _C * k * k
    bytes_accessed = 4 * (h_flat.size + sum(int(a.size) for a in args)
                          + N * _C * _ROWS * _COLS)

    out = pl.pallas_call(
        functools.partial(_conv_chain_kernel, ks=_CONV_KS),
        out_shape=jax.ShapeDtypeStruct((N, _C, _ROWS, _COLS), jnp.float32),
        grid_spec=pltpu.PrefetchScalarGridSpec(
            num_scalar_prefetch=0,
            grid=(N,),
            in_specs=[pl.BlockSpec((_C, 4096), lambda n: (n, 0))]
            + [pl.BlockSpec(a.shape, lambda n: (0, 0)) for a in args],
            out_specs=pl.BlockSpec((1, _C, _ROWS, _COLS), lambda n: (n, 0, 0, 0)),
        ),
        compiler_params=pltpu.CompilerParams(dimension_semantics=("parallel",)),
        cost_estimate=pl.CostEstimate(
            flops=flops, transcendentals=0, bytes_accessed=bytes_accessed),
    )(h_flat, *args)
    return out


@jax.jit
def _forward(x, lin_params, conv_params):
    h = _run_mlp(x, lin_params)
    return _run_convs(h, conv_params, x.shape[0])


def kernel(x, lin_w0, lin_b0, lin_w1, lin_b1, lin_w2, lin_b2, lin_w3, lin_b3,
           lin_w4, lin_b4, conv_w0, conv_b0, conv_w1, conv_b1, conv_w2,
           conv_b2, conv_w3, conv_b3, conv_w4, conv_b4, conv_w5, conv_b5):
    lin_params = [(lin_w0, lin_b0), (lin_w1, lin_b1), (lin_w2, lin_b2),
                  (lin_w3, lin_b3), (lin_w4, lin_b4)]
    conv_params = [(conv_w0, conv_b0), (conv_w1, conv_b1), (conv_w2, conv_b2),
                   (conv_w3, conv_b3), (conv_w4, conv_b4), (conv_w5, conv_b5)]
    return _forward(x, lin_params, conv_params)


# speedup vs baseline: 1.6317x; 1.0089x over previous
import functools

import jax
import jax.numpy as jnp
from jax.experimental import pallas as pl
from jax.experimental.pallas import tpu as pltpu

_CONV_KS = (7, 5, 3, 1, 1, 1)
_C = 16
_LIN_DIMS = (17, 64, 128, 256, 16, 1)

_ROWS, _COLS = 64, 64
_MR, _MC, _GUARD = 4, 8, 256
_WROW = _COLS + 2 * _MC
_HPAD = _ROWS + 2 * _MR
_L = _HPAD * _WROW
_LCAN = _L + 2 * _GUARD
_DSTART = _GUARD + _MR * _WROW
_LOUT = _ROWS * _WROW


def _mlp_kernel(x_ref, wp_ref, bp_ref, o_ref, *, nslab):
    D = x_ref.shape[1]
    P = x_ref.shape[2] * x_ref.shape[3]
    ws, bs = [], []
    off = 0
    for l in range(4):
        fin, fout = _LIN_DIMS[l], _LIN_DIMS[l + 1]
        ws.append(wp_ref[0:fout, off:off + fin])
        bs.append(bp_ref[0:fout, l:l + 1])
        off += max(128, fin)
    w4c = bp_ref[0:16, 4:5]
    b4s = bp_ref[0:1, 5:6]
    for s in range(nslab):
        h = x_ref[s].reshape(D, P)
        for w, b in zip(ws, bs):
            h = jnp.dot(w, h, preferred_element_type=jnp.float32) + b
            h = jnp.maximum(h, 0.0)
        o = jnp.sum(h * w4c, axis=0, keepdims=True) + b4s
        o_ref[s] = jnp.maximum(o, 0.0)[0]


def _conv_chain_kernel(img_ref, wp_ref, bp_ref, o_ref, *, ks):
    nlayer = len(ks)
    col = jax.lax.broadcasted_iota(jnp.int32, (1, _LOUT), 1) % _WROW
    colmask = jnp.logical_and(col >= _MC, col < _MC + _COLS)

    img = img_ref[...]
    z8 = jnp.zeros((_C, _MC), jnp.float32)
    pieces = []
    for r in range(_ROWS):
        pieces += [z8, img[:, r * _COLS:(r + 1) * _COLS], z8]
    flat = jnp.concatenate(pieces, axis=1)
    zg = jnp.zeros((_C, _DSTART), jnp.float32)
    can = jnp.concatenate([zg, flat, zg], axis=1)

    for layer in range(nlayer):
        k = ks[layer]
        p = (k - 1) // 2
        w_all = wp_ref[0:k * _C, 128 * layer:128 * layer + k * _C]
        b = bp_ref[:, layer:layer + 1]
        if k == 1:
            sl = can[:, _DSTART:_DSTART + _LOUT]
            acc = jnp.dot(w_all, sl, preferred_element_type=jnp.float32)
        else:
            xs = jnp.concatenate(
                [can[:, _GUARD + j - p:_GUARD + j - p + _L] for j in range(k)],
                axis=0)
            y = jnp.dot(w_all, xs, preferred_element_type=jnp.float32)
            base = _MR * _WROW
            acc = y[0 * _C:1 * _C, base + (0 - p) * _WROW:][:, :_LOUT]
            for i in range(1, k):
                off = base + (i - p) * _WROW
                acc = acc + y[i * _C:(i + 1) * _C, off:off + _LOUT]
        acc = jnp.maximum(acc + b, 0.0)
        if layer + 1 < nlayer:
            acc = jnp.where(colmask, acc, 0.0)
            can = jnp.concatenate([zg, acc, zg], axis=1)
        else:
            o_ref[0] = acc.reshape(_C, _ROWS, _WROW)[:, :, _MC:_MC + _COLS]


def _run_mlp(x, lin_params):
    N, C, D, H, W = x.shape
    P = H * W
    NC = N * C
    S = 8
    x_flat = x.reshape(NC, D, H, W)

    wsegs = []
    for l in range(4):
        w = lin_params[l][0]
        seg = max(128, w.shape[1])
        wsegs.append(jnp.pad(w, ((0, 256 - w.shape[0]), (0, seg - w.shape[1]))))
    wp = jnp.concatenate(wsegs, axis=1)
    bcols = []
    for l in range(4):
        b = lin_params[l][1]
        bcols.append(jnp.pad(b, (0, 256 - b.shape[0])).reshape(256, 1))
    w4, b4 = lin_params[4]
    bcols.append(jnp.pad(w4.reshape(-1), (0, 256 - 16)).reshape(256, 1))
    bcols.append(jnp.pad(b4, (0, 255)).reshape(256, 1))
    bp = jnp.concatenate(bcols, axis=1)

    flops = 2 * NC * P * sum(w.shape[0] * w.shape[1] for w, _ in lin_params)
    bytes_accessed = 4 * (x.size + wp.size + bp.size + NC * P)

    out = pl.pallas_call(
        functools.partial(_mlp_kernel, nslab=S),
        out_shape=jax.ShapeDtypeStruct((NC, P), jnp.float32),
        grid_spec=pltpu.PrefetchScalarGridSpec(
            num_scalar_prefetch=0,
            grid=(NC // S,),
            in_specs=[
                pl.BlockSpec((S, D, H, W), lambda g: (g, 0, 0, 0)),
                pl.BlockSpec(wp.shape, lambda g: (0, 0)),
                pl.BlockSpec(bp.shape, lambda g: (0, 0)),
            ],
            out_specs=pl.BlockSpec((S, P), lambda g: (g, 0)),
        ),
        compiler_params=pltpu.CompilerParams(dimension_semantics=("parallel",)),
        cost_estimate=pl.CostEstimate(
            flops=flops, transcendentals=0, bytes_accessed=bytes_accessed),
    )(x_flat, wp, bp)
    return out


def _run_convs(h_flat, conv_params, N):
    wsegs = []
    flops = 0
    for (w, b), k in zip(conv_params, _CONV_KS):
        w_all = jnp.transpose(w, (2, 0, 3, 1)).reshape(k * _C, k * _C)
        wsegs.append(jnp.pad(w_all, ((0, 112 - k * _C), (0, 128 - k * _C))))
        flops += 2 * N * _LOUT * _C * _C * k * k
    wp = jnp.concatenate(wsegs, axis=1)
    bp = jnp.stack([b for _, b in conv_params], axis=1)
    bytes_accessed = 4 * (h_flat.size + wp.size + bp.size
                          + N * _C * _ROWS * _COLS)

    out = pl.pallas_call(
        functools.partial(_conv_chain_kernel, ks=_CONV_KS),
        out_shape=jax.ShapeDtypeStruct((N, _C, _ROWS, _COLS), jnp.float32),
        grid_spec=pltpu.PrefetchScalarGridSpec(
            num_scalar_prefetch=0,
            grid=(N,),
            in_specs=[
                pl.BlockSpec((_C, 4096), lambda n: (n, 0)),
                pl.BlockSpec(wp.shape, lambda n: (0, 0)),
                pl.BlockSpec(bp.shape, lambda n: (0, 0)),
            ],
            out_specs=pl.BlockSpec((1, _C, _ROWS, _COLS), lambda n: (n, 0, 0, 0)),
        ),
        compiler_params=pltpu.CompilerParams(dimension_semantics=("parallel",)),
        cost_estimate=pl.CostEstimate(
            flops=flops, transcendentals=0, bytes_accessed=bytes_accessed),
    )(h_flat, wp, bp)
    return out


@jax.jit
def _forward(x, lin_params, conv_params):
    h = _run_mlp(x, lin_params)
    return _run_convs(h, conv_params, x.shape[0])


def kernel(x, lin_w0, lin_b0, lin_w1, lin_b1, lin_w2, lin_b2, lin_w3, lin_b3,
           lin_w4, lin_b4, conv_w0, conv_b0, conv_w1, conv_b1, conv_w2,
           conv_b2, conv_w3, conv_b3, conv_w4, conv_b4, conv_w5, conv_b5):
    lin_params = [(lin_w0, lin_b0), (lin_w1, lin_b1), (lin_w2, lin_b2),
                  (lin_w3, lin_b3), (lin_w4, lin_b4)]
    conv_params = [(conv_w0, conv_b0), (conv_w1, conv_b1), (conv_w2, conv_b2),
                   (conv_w3, conv_b3), (conv_w4, conv_b4), (conv_w5, conv_b5)]
    return _forward(x, lin_params, conv_params)
